# baseline (speedup 1.0000x reference)
import functools

import jax
import jax.numpy as jnp
from jax import lax
from jax.experimental import pallas as pl
from jax.experimental.pallas import tpu as pltpu
from jax.experimental.pallas import tpu_sc as plsc

_NBASIS = 1_000_000
_NUNIQUE = 25_000
_STRIDE = 31_248
_CHUNK = 31_312
_SIZES = (3920, 3920, 3920, 3920, 3920, 3920, 3920, 3872)
_OFFS = tuple(sum(_SIZES[:j]) for j in range(8))


def _sc_gather(step16, hmf, kinvidx):
    mesh = plsc.VectorSubcoreMesh(core_axis_name="c", subcore_axis_name="s")

    @functools.partial(
        pl.kernel,
        out_type=jax.ShapeDtypeStruct((_NBASIS,), jnp.float32),
        mesh=mesh,
        compiler_params=pltpu.CompilerParams(needs_layout_passes=False),
        scratch_types=[
            pltpu.VMEM((16,), jnp.float32),
            pltpu.VMEM((_NUNIQUE,), jnp.float32),
            pltpu.VMEM((_CHUNK,), jnp.int32),
            pltpu.VMEM((_CHUNK,), jnp.float32),
            pltpu.MemorySpace.VMEM_SHARED((_NUNIQUE,), jnp.float32),
            pltpu.SemaphoreType.DMA,
            pltpu.SemaphoreType.DMA,
            pltpu.SemaphoreType.DMA,
        ] + [pltpu.SemaphoreType.DMA] * 8,
    )
    def k(step_hbm, hmf_hbm, idx_hbm, out_hbm,
          step_v, table_v, idx_v, out_v, table_sp,
          sem_t, sem_s, sem_o, *sem_i):
        sid = lax.axis_index("s")
        wid = sid * 2 + lax.axis_index("c")
        base = wid * _STRIDE

        cp_i = [
            pltpu.async_copy(
                idx_hbm.at[pl.ds(base + _OFFS[j], _SIZES[j])],
                idx_v.at[pl.ds(_OFFS[j], _SIZES[j])],
                sem_i[j],
            )
            for j in range(8)
        ]
        cp_s = pltpu.async_copy(step_hbm, step_v, sem_s)

        @pl.when(sid == 0)
        def _():
            pltpu.sync_copy(hmf_hbm, table_sp)

        plsc.subcore_barrier()
        cp_t = pltpu.async_copy(table_sp, table_v, sem_t)
        cp_s.wait()
        step_vec = step_v[...]
        cp_t.wait()

        cp_o = []
        for j in range(8):
            cp_i[j].wait()

            @plsc.parallel_loop(_OFFS[j], _OFFS[j] + _SIZES[j], 16, unroll=8)
            def body(off):
                idx16 = idx_v[pl.ds(off, 16)]
                vals = plsc.load_gather(table_v, [idx16])
                out_v[pl.ds(off, 16)] = vals * step_vec

            cp_o.append(
                pltpu.async_copy(
                    out_v.at[pl.ds(_OFFS[j], _SIZES[j])],
                    out_hbm.at[pl.ds(base + _OFFS[j], _SIZES[j])],
                    sem_o,
                )
            )
        for cp in cp_o:
            cp.wait()

    return k(step16, hmf, kinvidx)


def kernel(step, hmf, kinvidx):
    step16 = jnp.full((16,), step, dtype=jnp.float32)
    idx = kinvidx.astype(jnp.int32)
    return _sc_gather(step16, hmf, idx)


# speedup vs baseline: 1.0289x; 1.0289x over previous
import functools

import jax
import jax.numpy as jnp
from jax import lax
from jax.experimental import pallas as pl
from jax.experimental.pallas import tpu as pltpu
from jax.experimental.pallas import tpu_sc as plsc

_NBASIS = 1_000_000
_NUNIQUE = 25_000
_STRIDE = 31_248
_CHUNK = 31_312
_SIZES = (7840, 7840, 7840, 7792)
_OFFS = (0, 7840, 15680, 23520)


def _sc_gather(step16, hmf, kinvidx):
    mesh = plsc.VectorSubcoreMesh(core_axis_name="c", subcore_axis_name="s")

    @functools.partial(
        pl.kernel,
        out_type=jax.ShapeDtypeStruct((_NBASIS,), jnp.float32),
        mesh=mesh,
        compiler_params=pltpu.CompilerParams(needs_layout_passes=False),
        scratch_types=[
            pltpu.VMEM((16,), jnp.float32),
            pltpu.VMEM((_NUNIQUE,), jnp.float32),
            pltpu.VMEM((_CHUNK,), jnp.int32),
            pltpu.VMEM((_CHUNK,), jnp.float32),
            pltpu.MemorySpace.VMEM_SHARED((_NUNIQUE,), jnp.float32),
            pltpu.SemaphoreType.DMA,
            pltpu.SemaphoreType.DMA,
            pltpu.SemaphoreType.DMA,
        ] + [pltpu.SemaphoreType.DMA] * 4,
    )
    def k(step_hbm, hmf_hbm, idx_hbm, out_hbm,
          step_v, table_v, idx_v, out_v, table_sp,
          sem_t, sem_s, sem_o, *sem_i):
        sid = lax.axis_index("s")
        wid = sid * 2 + lax.axis_index("c")
        base = wid * _STRIDE

        cp_i = [
            pltpu.async_copy(
                idx_hbm.at[pl.ds(base + _OFFS[j], _SIZES[j])],
                idx_v.at[pl.ds(_OFFS[j], _SIZES[j])],
                sem_i[j],
            )
            for j in range(4)
        ]
        cp_s = pltpu.async_copy(step_hbm, step_v, sem_s)

        @pl.when(sid == 0)
        def _():
            pltpu.sync_copy(hmf_hbm, table_sp)

        plsc.subcore_barrier()
        cp_t = pltpu.async_copy(table_sp, table_v, sem_t)
        cp_s.wait()
        step_vec = step_v[...]
        cp_t.wait()

        cp_o = []
        for j in range(4):
            cp_i[j].wait()

            @plsc.parallel_loop(_OFFS[j], _OFFS[j] + _SIZES[j], 16, unroll=8)
            def body(off):
                idx16 = idx_v[pl.ds(off, 16)]
                vals = plsc.load_gather(table_v, [idx16])
                out_v[pl.ds(off, 16)] = vals * step_vec

            cp_o.append(
                pltpu.async_copy(
                    out_v.at[pl.ds(_OFFS[j], _SIZES[j])],
                    out_hbm.at[pl.ds(base + _OFFS[j], _SIZES[j])],
                    sem_o,
                )
            )
        for cp in cp_o:
            cp.wait()

    return k(step16, hmf, kinvidx)


def kernel(step, hmf, kinvidx):
    step16 = jnp.full((16,), step, dtype=jnp.float32)
    idx = kinvidx.astype(jnp.int32)
    return _sc_gather(step16, hmf, idx)


# speedup vs baseline: 1.0396x; 1.0104x over previous
import functools

import jax
import jax.numpy as jnp
from jax import lax
from jax.experimental import pallas as pl
from jax.experimental.pallas import tpu as pltpu
from jax.experimental.pallas import tpu_sc as plsc

_NBASIS = 1_000_000
_NUNIQUE = 25_000
_STRIDE = 31_248
_CHUNK = 31_312
_SIZES = (15664, 15648)
_OFFS = (0, 15664)


def _sc_gather(step16, hmf, kinvidx):
    mesh = plsc.VectorSubcoreMesh(core_axis_name="c", subcore_axis_name="s")

    @functools.partial(
        pl.kernel,
        out_type=jax.ShapeDtypeStruct((_NBASIS,), jnp.float32),
        mesh=mesh,
        compiler_params=pltpu.CompilerParams(needs_layout_passes=False),
        scratch_types=[
            pltpu.VMEM((16,), jnp.float32),
            pltpu.VMEM((_NUNIQUE,), jnp.float32),
            pltpu.VMEM((_CHUNK,), jnp.int32),
            pltpu.VMEM((_CHUNK,), jnp.float32),
            pltpu.MemorySpace.VMEM_SHARED((_NUNIQUE,), jnp.float32),
            pltpu.SemaphoreType.DMA,
            pltpu.SemaphoreType.DMA,
            pltpu.SemaphoreType.DMA,
        ] + [pltpu.SemaphoreType.DMA] * 2,
    )
    def k(step_hbm, hmf_hbm, idx_hbm, out_hbm,
          step_v, table_v, idx_v, out_v, table_sp,
          sem_t, sem_s, sem_o, *sem_i):
        sid = lax.axis_index("s")
        wid = sid * 2 + lax.axis_index("c")
        base = wid * _STRIDE

        cp_i = [
            pltpu.async_copy(
                idx_hbm.at[pl.ds(base + _OFFS[j], _SIZES[j])],
                idx_v.at[pl.ds(_OFFS[j], _SIZES[j])],
                sem_i[j],
            )
            for j in range(2)
        ]
        cp_s = pltpu.async_copy(step_hbm, step_v, sem_s)

        @pl.when(sid == 0)
        def _():
            pltpu.sync_copy(hmf_hbm, table_sp)

        plsc.subcore_barrier()
        cp_t = pltpu.async_copy(table_sp, table_v, sem_t)
        cp_s.wait()
        step_vec = step_v[...]
        cp_t.wait()

        cp_o = []
        for j in range(2):
            cp_i[j].wait()

            @plsc.parallel_loop(_OFFS[j], _OFFS[j] + _SIZES[j], 16, unroll=8)
            def body(off):
                idx16 = idx_v[pl.ds(off, 16)]
                vals = plsc.load_gather(table_v, [idx16])
                out_v[pl.ds(off, 16)] = vals * step_vec

            cp_o.append(
                pltpu.async_copy(
                    out_v.at[pl.ds(_OFFS[j], _SIZES[j])],
                    out_hbm.at[pl.ds(base + _OFFS[j], _SIZES[j])],
                    sem_o,
                )
            )
        for cp in cp_o:
            cp.wait()

    return k(step16, hmf, kinvidx)


def kernel(step, hmf, kinvidx):
    step16 = jnp.full((16,), step, dtype=jnp.float32)
    idx = kinvidx.astype(jnp.int32)
    return _sc_gather(step16, hmf, idx)
